# Initial kernel scaffold; baseline (speedup 1.0000x reference)
import jax
import jax.numpy as jnp
from jax.experimental import pallas as pl


def kernel(logits, targets):
    raise NotImplementedError("write your pallas kernel here")


# speedup vs baseline: 7.6082x; 7.6082x over previous
import jax
import jax.numpy as jnp
from jax import lax
from jax.experimental import pallas as pl
from jax.experimental.pallas import tpu as pltpu
from jax.experimental.pallas import tpu_sc as plsc

B_BITS = 15
NB = 1 << B_BITS
HIST = 2 * NB
N_IMG = 8
NPIX = 512 * 512
NC, NS = 2, 16
IMGS_PER_SC = N_IMG // NC
CHUNK = NPIX // NS
ROWS = CHUNK // 128
ZBUF = NB // 4
SC_HIST = IMGS_PER_SC * HIST
ZSLICE = SC_HIST // NS


def _sc_hist_body(lg_hbm, tg_hbm, cnt_out, sum_out,
                  lg_v, tg_v, idx_v, val_v, ones_v, zero_v, cnt_sh, sum_sh):
    c = lax.axis_index("c")
    s = lax.axis_index("s")

    def _zb(i, _):
        zero_v[pl.ds(i * 16, 16)] = jnp.zeros((16,), jnp.float32)
        return 0
    lax.fori_loop(0, ZBUF // 16, _zb, 0)
    for j in range(8):
        ones_v[pl.ds(j * 16, 16)] = jnp.ones((16,), jnp.float32)

    for k in range(ZSLICE // ZBUF):
        off = s * ZSLICE + k * ZBUF
        pltpu.sync_copy(zero_v, cnt_sh.at[pl.ds(off, ZBUF)])
        pltpu.sync_copy(zero_v, sum_sh.at[pl.ds(off, ZBUF)])
    plsc.subcore_barrier()

    for il in range(IMGS_PER_SC):
        img = c * IMGS_PER_SC + il
        base = img * NPIX + s * CHUNK
        pltpu.sync_copy(lg_hbm.at[pl.ds(base, CHUNK)], lg_v)
        pltpu.sync_copy(tg_hbm.at[pl.ds(base, CHUNK)], tg_v)

        def _row(row, _, il=il):
            for jj in range(8):
                i16 = (row * 8 + jj) * 16
                l = lg_v[pl.ds(i16, 16)]
                t = tg_v[pl.ds(i16, 16)]
                tb = t > 0.5
                sign = jnp.where(tb, 1.0, -1.0).astype(jnp.float32)
                r = jnp.maximum(1.0 - l * sign, 0.0).astype(jnp.float32)
                b = lax.shift_right_logical(
                    lax.bitcast_convert_type(r, jnp.int32), 32 - B_BITS)
                idx = b + jnp.where(tb, NB, 0).astype(jnp.int32) + il * HIST
                idx_v[row, pl.ds(jj * 16, 16)] = idx
                val_v[row, pl.ds(jj * 16, 16)] = r
            return 0
        lax.fori_loop(0, ROWS, _row, 0)

        def _scat(j, _):
            pltpu.sync_copy(ones_v, cnt_sh.at[idx_v.at[j]], add=True)
            pltpu.sync_copy(val_v.at[j], sum_sh.at[idx_v.at[j]], add=True)
            return 0
        lax.fori_loop(0, ROWS, _scat, 0)

    plsc.subcore_barrier()
    off_sh = s * ZSLICE
    off_out = c * SC_HIST + s * ZSLICE
    pltpu.sync_copy(cnt_sh.at[pl.ds(off_sh, ZSLICE)],
                    cnt_out.at[pl.ds(off_out, ZSLICE)])
    pltpu.sync_copy(sum_sh.at[pl.ds(off_sh, ZSLICE)],
                    sum_out.at[pl.ds(off_out, ZSLICE)])


import functools


@functools.lru_cache(maxsize=None)
def _build_sc_hist():
  return pl.kernel(
    _sc_hist_body,
    out_type=(jax.ShapeDtypeStruct((N_IMG * HIST,), jnp.float32),
              jax.ShapeDtypeStruct((N_IMG * HIST,), jnp.float32)),
    mesh=plsc.VectorSubcoreMesh(core_axis_name="c", subcore_axis_name="s",
                                num_cores=NC, num_subcores=NS),
    scratch_types=[
        pltpu.VMEM((CHUNK,), jnp.float32),
        pltpu.VMEM((CHUNK,), jnp.float32),
        pltpu.VMEM((ROWS, 128), jnp.int32),
        pltpu.VMEM((ROWS, 128), jnp.float32),
        pltpu.VMEM((128,), jnp.float32),
        pltpu.VMEM((ZBUF,), jnp.float32),
        pltpu.VMEM_SHARED((SC_HIST,), jnp.float32),
        pltpu.VMEM_SHARED((SC_HIST,), jnp.float32),
    ],
  )


_R, _C = NB // 128, 128


def _suffix_excl(x, upper_incl, strict_lower):
    lane_cum = jax.lax.dot_general(
        x, upper_incl, (((1,), (0,)), ((), ())),
        preferred_element_type=jnp.float32)
    row_prev = jax.lax.dot_general(
        strict_lower, x, (((1,), (0,)), ((), ())),
        preferred_element_type=jnp.float32)
    prefix_incl = lane_cum + jnp.sum(row_prev, axis=1, keepdims=True)
    return jnp.sum(x) - prefix_incl


def _tc_finish_body(cnt_ref, sum_ref, out_ref):
    i = pl.program_id(0)

    @pl.when(i == 0)
    def _():
        out_ref[...] = jnp.zeros((1, 1), jnp.float32)

    cn = cnt_ref[0, 0]
    cp = cnt_ref[0, 1]
    sn = sum_ref[0, 0]
    sp = sum_ref[0, 1]

    ci = lax.broadcasted_iota(jnp.int32, (128, 128), 0)
    cj = lax.broadcasted_iota(jnp.int32, (128, 128), 1)
    upper_incl = (ci <= cj).astype(jnp.float32)
    ri = lax.broadcasted_iota(jnp.int32, (_R, _R), 0)
    rj = lax.broadcasted_iota(jnp.int32, (_R, _R), 1)
    strict_lower = (rj < ri).astype(jnp.float32)

    p = jnp.sum(cp)
    q_ab = _suffix_excl(cp, upper_incl, strict_lower)
    m_ab = _suffix_excl(cn, upper_incl, strict_lower)

    d0 = jnp.maximum(p + m_ab, 1.0)
    d1 = jnp.maximum(p + m_ab + cn, 1.0)
    pos_term = jnp.sum(sp / d0)
    coef = (p - q_ab - cp) * (1.0 / d0 - 1.0 / d1) / jnp.maximum(cn, 1.0)
    neg_term = jnp.sum(sn * coef)
    loss = pos_term + neg_term

    flat = (lax.broadcasted_iota(jnp.int32, (_R, _C), 0) * _C
            + lax.broadcasted_iota(jnp.int32, (_R, _C), 1))
    occupied = cn > 0.0
    bmax = jnp.max(jnp.where(occupied, flat, -1))
    loss0 = jnp.sum(jnp.where(flat == bmax, sn / jnp.maximum(cn, 1.0), 0.0))
    loss = jnp.where(p > 0.0, loss, loss0)

    out_ref[...] += (loss / N_IMG).reshape(1, 1)


_tc_finish = pl.pallas_call(
    _tc_finish_body,
    grid=(N_IMG,),
    in_specs=[
        pl.BlockSpec((1, 2, _R, _C), lambda i: (i, 0, 0, 0)),
        pl.BlockSpec((1, 2, _R, _C), lambda i: (i, 0, 0, 0)),
    ],
    out_specs=pl.BlockSpec((1, 1), lambda i: (0, 0)),
    out_shape=jax.ShapeDtypeStruct((1, 1), jnp.float32),
)


def kernel(logits, targets):
    lg = logits.reshape(N_IMG * NPIX)
    tg = targets.reshape(N_IMG * NPIX)
    cnt, sm = _build_sc_hist()(lg, tg)
    cnt = cnt.reshape(N_IMG, 2, _R, _C)
    sm = sm.reshape(N_IMG, 2, _R, _C)
    out = _tc_finish(cnt, sm)
    return out[0, 0]
